# baseline (device time: 35875 ns/iter reference)
import jax
import jax.numpy as jnp
from jax import lax
from jax.experimental import pallas as pl
from jax.experimental.pallas import tpu as pltpu

N_ROWS = 1024
HALF = N_ROWS // 2
C = 16
R = HALF // C


def kernel(x, pi):
    shard_shape = x.shape

    def body(pi_ref, x_ref, out_ref, xs_sems, xr_sems, ys_sems, yr_sems,
             local_sem):
        my_x = lax.axis_index("x")
        my_y = lax.axis_index("y")
        dst_x = pi_ref[my_x]

        @pl.when(dst_x == my_x)
        def _local():
            cp = pltpu.make_async_copy(x_ref, out_ref, local_sem)
            cp.start()
            cp.wait()

        @pl.when(dst_x != my_x)
        def _swap():
            barrier = pltpu.get_barrier_semaphore()
            for nbr in ((dst_x, my_y), (my_x, 1 - my_y)):
                pl.semaphore_signal(
                    barrier, inc=1, device_id=nbr,
                    device_id_type=pl.DeviceIdType.MESH,
                )
            pl.semaphore_wait(barrier, 2)

            half0 = my_y * HALF

            x_sends = []
            for c in range(C):
                rows = pl.ds(half0 + c * R, R)
                rdma = pltpu.make_async_remote_copy(
                    src_ref=x_ref.at[0, rows, :],
                    dst_ref=out_ref.at[0, rows, :],
                    send_sem=xs_sems.at[c],
                    recv_sem=xr_sems.at[c],
                    device_id=(dst_x, my_y),
                    device_id_type=pl.DeviceIdType.MESH,
                )
                rdma.start()
                x_sends.append(rdma)

            y_sends = []
            for c in range(C):
                x_sends[c].wait_recv()
                rows = pl.ds(half0 + c * R, R)
                fwd = pltpu.make_async_remote_copy(
                    src_ref=out_ref.at[0, rows, :],
                    dst_ref=out_ref.at[0, rows, :],
                    send_sem=ys_sems.at[c],
                    recv_sem=yr_sems.at[c],
                    device_id=(my_x, 1 - my_y),
                    device_id_type=pl.DeviceIdType.MESH,
                )
                fwd.start()
                y_sends.append(fwd)

            other0 = (1 - my_y) * HALF
            for c in range(C):
                rows = pl.ds(other0 + c * R, R)
                recv = pltpu.make_async_remote_copy(
                    src_ref=out_ref.at[0, rows, :],
                    dst_ref=out_ref.at[0, rows, :],
                    send_sem=ys_sems.at[c],
                    recv_sem=yr_sems.at[c],
                    device_id=(my_x, 1 - my_y),
                    device_id_type=pl.DeviceIdType.MESH,
                )
                recv.wait_recv()
            for c in range(C):
                x_sends[c].wait_send()
                y_sends[c].wait_send()

    grid_spec = pltpu.PrefetchScalarGridSpec(
        num_scalar_prefetch=1,
        in_specs=[pl.BlockSpec(memory_space=pltpu.MemorySpace.HBM)],
        out_specs=pl.BlockSpec(memory_space=pltpu.MemorySpace.HBM),
        scratch_shapes=[
            pltpu.SemaphoreType.DMA((C,)),
            pltpu.SemaphoreType.DMA((C,)),
            pltpu.SemaphoreType.DMA((C,)),
            pltpu.SemaphoreType.DMA((C,)),
            pltpu.SemaphoreType.DMA,
        ],
    )

    return pl.pallas_call(
        body,
        grid_spec=grid_spec,
        out_shape=jax.ShapeDtypeStruct(shard_shape, jnp.float32),
        compiler_params=pltpu.CompilerParams(collective_id=0),
    )(pi, x)


# device time: 35785 ns/iter; 1.0025x vs baseline; 1.0025x over previous
import jax
import jax.numpy as jnp
from jax import lax
from jax.experimental import pallas as pl
from jax.experimental.pallas import tpu as pltpu

N_ROWS = 1024
HALF = N_ROWS // 2
C = 32
R = HALF // C


def kernel(x, pi):
    shard_shape = x.shape

    def body(pi_ref, x_ref, out_ref, xs_sems, xr_sems, ys_sems, yr_sems,
             local_sem):
        my_x = lax.axis_index("x")
        my_y = lax.axis_index("y")
        dst_x = pi_ref[my_x]

        @pl.when(dst_x == my_x)
        def _local():
            cp = pltpu.make_async_copy(x_ref, out_ref, local_sem)
            cp.start()
            cp.wait()

        @pl.when(dst_x != my_x)
        def _swap():
            barrier = pltpu.get_barrier_semaphore()
            for nbr in ((dst_x, my_y), (my_x, 1 - my_y)):
                pl.semaphore_signal(
                    barrier, inc=1, device_id=nbr,
                    device_id_type=pl.DeviceIdType.MESH,
                )
            pl.semaphore_wait(barrier, 2)

            half0 = my_y * HALF

            x_sends = []
            for c in range(C):
                rows = pl.ds(half0 + c * R, R)
                rdma = pltpu.make_async_remote_copy(
                    src_ref=x_ref.at[0, rows, :],
                    dst_ref=out_ref.at[0, rows, :],
                    send_sem=xs_sems.at[c],
                    recv_sem=xr_sems.at[c],
                    device_id=(dst_x, my_y),
                    device_id_type=pl.DeviceIdType.MESH,
                )
                rdma.start()
                x_sends.append(rdma)

            y_sends = []
            for c in range(C):
                x_sends[c].wait_recv()
                rows = pl.ds(half0 + c * R, R)
                fwd = pltpu.make_async_remote_copy(
                    src_ref=out_ref.at[0, rows, :],
                    dst_ref=out_ref.at[0, rows, :],
                    send_sem=ys_sems.at[c],
                    recv_sem=yr_sems.at[c],
                    device_id=(my_x, 1 - my_y),
                    device_id_type=pl.DeviceIdType.MESH,
                )
                fwd.start()
                y_sends.append(fwd)

            other0 = (1 - my_y) * HALF
            for c in range(C):
                rows = pl.ds(other0 + c * R, R)
                recv = pltpu.make_async_remote_copy(
                    src_ref=out_ref.at[0, rows, :],
                    dst_ref=out_ref.at[0, rows, :],
                    send_sem=ys_sems.at[c],
                    recv_sem=yr_sems.at[c],
                    device_id=(my_x, 1 - my_y),
                    device_id_type=pl.DeviceIdType.MESH,
                )
                recv.wait_recv()
            for c in range(C):
                x_sends[c].wait_send()
                y_sends[c].wait_send()

    grid_spec = pltpu.PrefetchScalarGridSpec(
        num_scalar_prefetch=1,
        in_specs=[pl.BlockSpec(memory_space=pltpu.MemorySpace.HBM)],
        out_specs=pl.BlockSpec(memory_space=pltpu.MemorySpace.HBM),
        scratch_shapes=[
            pltpu.SemaphoreType.DMA((C,)),
            pltpu.SemaphoreType.DMA((C,)),
            pltpu.SemaphoreType.DMA((C,)),
            pltpu.SemaphoreType.DMA((C,)),
            pltpu.SemaphoreType.DMA,
        ],
    )

    return pl.pallas_call(
        body,
        grid_spec=grid_spec,
        out_shape=jax.ShapeDtypeStruct(shard_shape, jnp.float32),
        compiler_params=pltpu.CompilerParams(collective_id=0),
    )(pi, x)
